# baseline (speedup 1.0000x reference)
import functools

import jax
import jax.numpy as jnp
from jax import lax
from jax.experimental import pallas as pl
from jax.experimental.pallas import tpu as pltpu
from jax.experimental.pallas import tpu_sc as plsc

N_ROWS = 16384
D = 1024
LANES = 16
VPR = D // LANES
NW = 32
ROWS_PER_W = N_ROWS // NW
C = 16
NCHUNK = ROWS_PER_W // C
NBX = 3
NBP = 4


def _make_sc_kernel():
    mesh = plsc.VectorSubcoreMesh(core_axis_name="c", subcore_axis_name="s")

    @functools.partial(
        pl.kernel,
        mesh=mesh,
        out_type=jax.ShapeDtypeStruct((N_ROWS, D), jnp.float32),
        scratch_types=(
            [pltpu.VMEM((NCHUNK, C), jnp.int32)]
            + [pltpu.VMEM((C, D), jnp.float32)] * (NBX + NBP)
            + [pltpu.SemaphoreType.DMA] * (2 * NBX + NBP + 1)
        ),
    )
    def sc_kernel(x_hbm, t2_hbm, pe_hbm, out_hbm, idx_all, *rest):
        xb = list(rest[0:NBX])
        pb = list(rest[NBX:NBX + NBP])
        sems = rest[NBX + NBP:]
        sx = sems[0:NBX]
        sg = sems[NBX:NBX + NBP]
        ss = sems[NBX + NBP:2 * NBX + NBP]
        sidx = sems[2 * NBX + NBP]

        wid = lax.axis_index("s") * 2 + lax.axis_index("c")
        base = wid * ROWS_PER_W
        cbase = wid * NCHUNK

        idx_cp = pltpu.async_copy(t2_hbm.at[pl.ds(cbase, NCHUNK)], idx_all,
                                  sidx)

        xls = [None] * NBX
        gls = [None] * NBP
        stores = [None] * NBX

        def start_x(i):
            b = i % NBX
            return pltpu.async_copy(x_hbm.at[pl.ds(base + i * C, C)],
                                    xb[b], sx[b])

        def start_gather(i):
            b = i % NBP
            return pltpu.async_copy(pe_hbm.at[idx_all.at[i]], pb[b], sg[b])

        for i in range(NBX - 1):
            xls[i] = start_x(i)
        idx_cp.wait()
        for i in range(NBP - 1):
            gls[i] = start_gather(i)

        for i in range(NCHUNK):
            bx, bp = i % NBX, i % NBP
            jx = i + NBX - 1
            if jx < NCHUNK:
                nb = jx % NBX
                if stores[nb] is not None:
                    stores[nb].wait()
                    stores[nb] = None
                xls[nb] = start_x(jx)
            jg = i + NBP - 1
            if jg < NCHUNK:
                gls[jg % NBP] = start_gather(jg)
            xls[bx].wait()
            gls[bp].wait()

            @plsc.parallel_loop(0, C * VPR, unroll=8)
            def add(k, _b=bx, _p=bp):
                r = k // VPR
                col = (k % VPR) * LANES
                plsc.addupdate(xb[_b].at[r, pl.ds(col, LANES)],
                               pb[_p][r, pl.ds(col, LANES)])

            stores[bx] = pltpu.async_copy(
                xb[bx], out_hbm.at[pl.ds(base + i * C, C)], ss[bx])

        for b in range(NBX):
            if stores[b] is not None:
                stores[b].wait()
                stores[b] = None

    return sc_kernel


def kernel(x, times, pe):
    B, L, _ = x.shape
    xf = x.reshape(N_ROWS, D)
    tf = times.reshape(N_ROWS // C, C)
    out = _make_sc_kernel()(xf, tf, pe)
    return out.reshape(B, L, D)


# speedup vs baseline: 1.0033x; 1.0033x over previous
import functools

import jax
import jax.numpy as jnp
from jax import lax
from jax.experimental import pallas as pl
from jax.experimental.pallas import tpu as pltpu
from jax.experimental.pallas import tpu_sc as plsc

N_ROWS = 16384
D = 1024
LANES = 16
VPR = D // LANES
NW = 32
ROWS_PER_W = N_ROWS // NW
C = 16
NCHUNK = ROWS_PER_W // C
NBX = 3
NBP = 3


def _make_sc_kernel():
    mesh = plsc.VectorSubcoreMesh(core_axis_name="c", subcore_axis_name="s")

    @functools.partial(
        pl.kernel,
        mesh=mesh,
        out_type=jax.ShapeDtypeStruct((N_ROWS, D), jnp.float32),
        scratch_types=(
            [pltpu.VMEM((NCHUNK, C), jnp.int32)]
            + [pltpu.VMEM((C, D), jnp.float32)] * (NBX + NBP)
            + [pltpu.SemaphoreType.DMA] * (2 * NBX + NBP + 1)
        ),
    )
    def sc_kernel(x_hbm, t2_hbm, pe_hbm, out_hbm, idx_all, *rest):
        xb = list(rest[0:NBX])
        pb = list(rest[NBX:NBX + NBP])
        sems = rest[NBX + NBP:]
        sx = sems[0:NBX]
        sg = sems[NBX:NBX + NBP]
        ss = sems[NBX + NBP:2 * NBX + NBP]
        sidx = sems[2 * NBX + NBP]

        wid = lax.axis_index("s") * 2 + lax.axis_index("c")
        base = wid * ROWS_PER_W
        cbase = wid * NCHUNK

        idx_cp = pltpu.async_copy(t2_hbm.at[pl.ds(cbase, NCHUNK)], idx_all,
                                  sidx)

        xls = [None] * NBX
        gls = [None] * NBP
        stores = [None] * NBX

        def start_x(i):
            b = i % NBX
            return pltpu.async_copy(x_hbm.at[pl.ds(base + i * C, C)],
                                    xb[b], sx[b])

        def start_gather(i):
            b = i % NBP
            return pltpu.async_copy(pe_hbm.at[idx_all.at[i]], pb[b], sg[b])

        for i in range(NBX - 1):
            xls[i] = start_x(i)
        idx_cp.wait()
        for i in range(NBP - 1):
            gls[i] = start_gather(i)

        for i in range(NCHUNK):
            bx, bp = i % NBX, i % NBP
            jx = i + NBX - 1
            if jx < NCHUNK:
                nb = jx % NBX
                if stores[nb] is not None:
                    stores[nb].wait()
                    stores[nb] = None
                xls[nb] = start_x(jx)
            jg = i + NBP - 1
            if jg < NCHUNK:
                gls[jg % NBP] = start_gather(jg)
            xls[bx].wait()
            gls[bp].wait()

            @plsc.parallel_loop(0, C * VPR, unroll=8)
            def add(k, _b=bx, _p=bp):
                r = k // VPR
                col = (k % VPR) * LANES
                plsc.addupdate(xb[_b].at[r, pl.ds(col, LANES)],
                               pb[_p][r, pl.ds(col, LANES)])

            stores[bx] = pltpu.async_copy(
                xb[bx], out_hbm.at[pl.ds(base + i * C, C)], ss[bx])

        for b in range(NBX):
            if stores[b] is not None:
                stores[b].wait()
                stores[b] = None

    return sc_kernel


def kernel(x, times, pe):
    B, L, _ = x.shape
    xf = x.reshape(N_ROWS, D)
    tf = times.reshape(N_ROWS // C, C)
    out = _make_sc_kernel()(xf, tf, pe)
    return out.reshape(B, L, D)
